# baseline (speedup 1.0000x reference)
import jax
import jax.numpy as jnp
from jax.experimental import pallas as pl
from jax.experimental.pallas import tpu as pltpu

_N = 4
_P = 4096
_BC = 1024
_NC = _P // _BC
_L = 128


def _chamfer_kernel(p1_ref, p2_ref, out_ref):
    def batch_body(n, acc):
        a = p1_ref[n]
        b = p2_ref[n]
        a2 = jnp.sum(a * a, axis=1, keepdims=True)
        b2r = jnp.sum(b * b, axis=1, keepdims=True).T

        tot = jnp.zeros((1, 1), jnp.float32)
        rowm = None
        for c in range(_NC):
            bc = -2.0 * b[c * _BC:(c + 1) * _BC, :]
            ab = jax.lax.dot_general(
                a, bc, (((1,), (1,)), ((), ())),
                preferred_element_type=jnp.float32,
            )
            for k in range(_BC // _L):
                j0 = c * _BC + k * _L
                t = b2r[:, j0:j0 + _L] + ab[:, k * _L:(k + 1) * _L]
                rowm = t if rowm is None else jnp.minimum(rowm, t)
            colc = jnp.min(a2 + ab, axis=0, keepdims=True)
            tot += jnp.sum(colc + b2r[:, c * _BC:(c + 1) * _BC])[None, None]

        rowfin = jnp.min(rowm, axis=1, keepdims=True)
        tot += jnp.sum(rowfin + a2)[None, None]
        return acc + tot

    acc = jax.lax.fori_loop(0, _N, batch_body, jnp.zeros((1, 1), jnp.float32))
    out_ref[...] = acc[0, 0] * (1.0 / (_N * _P))


@jax.jit
def kernel(p1, p2):
    p1p = p1
    p2p = p2

    out = pl.pallas_call(
        _chamfer_kernel,
        out_specs=pl.BlockSpec(memory_space=pltpu.SMEM),
        out_shape=jax.ShapeDtypeStruct((), jnp.float32),
        compiler_params=pltpu.CompilerParams(
            vmem_limit_bytes=100 * 1024 * 1024,
        ),
    )(p1p, p2p)
    return out


# speedup vs baseline: 1.0444x; 1.0444x over previous
import jax
import jax.numpy as jnp
from jax.experimental import pallas as pl
from jax.experimental.pallas import tpu as pltpu

_N = 4
_P = 4096
_BC = 256
_NC = _P // _BC
_L = 128


def _chamfer_kernel(p1_ref, p2_ref, out_ref, acc_ref):
    n = pl.program_id(0)

    a = p1_ref[0]
    b = p2_ref[0]
    a2 = jnp.sum(a * a, axis=1, keepdims=True)
    b2r = jnp.sum(b * b, axis=1, keepdims=True).T

    tot = jnp.zeros((1, 1), jnp.float32)
    rowm = None
    for c in range(_NC):
        bc = -2.0 * b[c * _BC:(c + 1) * _BC, :]
        ab = jax.lax.dot_general(
            a, bc, (((1,), (1,)), ((), ())),
            preferred_element_type=jnp.float32,
        )
        for k in range(_BC // _L):
            j0 = c * _BC + k * _L
            t = b2r[:, j0:j0 + _L] + ab[:, k * _L:(k + 1) * _L]
            rowm = t if rowm is None else jnp.minimum(rowm, t)
        colc = jnp.min(a2 + ab, axis=0, keepdims=True)
        tot += jnp.sum(colc + b2r[:, c * _BC:(c + 1) * _BC])[None, None]

    rowfin = jnp.min(rowm, axis=1, keepdims=True)
    tot += jnp.sum(rowfin + a2)[None, None]

    @pl.when(n == 0)
    def _init():
        acc_ref[...] = jnp.zeros((1, 1), jnp.float32)

    acc_ref[...] += tot

    @pl.when(n == _N - 1)
    def _fin():
        out_ref[...] = acc_ref[...] * (1.0 / (_N * _P))


@jax.jit
def kernel(p1, p2):
    p1p = p1
    p2p = p2

    out = pl.pallas_call(
        _chamfer_kernel,
        grid=(_N,),
        in_specs=[
            pl.BlockSpec((1, _P, 3), lambda n: (n, 0, 0)),
            pl.BlockSpec((1, _P, 3), lambda n: (n, 0, 0)),
        ],
        out_specs=pl.BlockSpec((1, 1), lambda n: (0, 0)),
        out_shape=jax.ShapeDtypeStruct((1, 1), jnp.float32),
        scratch_shapes=[
            pltpu.VMEM((1, 1), jnp.float32),
        ],
        compiler_params=pltpu.CompilerParams(
            vmem_limit_bytes=100 * 1024 * 1024,
        ),
    )(p1p, p2p)
    return out[0, 0]


# speedup vs baseline: 1.0535x; 1.0087x over previous
import jax
import jax.numpy as jnp
from jax.experimental import pallas as pl
from jax.experimental.pallas import tpu as pltpu

_N = 4
_P = 4096
_BC = 512
_NC = _P // _BC
_L = 128


def _chamfer_kernel(p1_ref, p2_ref, out_ref, acc_ref):
    n = pl.program_id(0)

    a = p1_ref[0]
    b = p2_ref[0]
    a2 = jnp.sum(a * a, axis=1, keepdims=True)
    b2r = jnp.sum(b * b, axis=1, keepdims=True).T

    tot = jnp.zeros((1, 1), jnp.float32)
    rowm = None
    for c in range(_NC):
        bc = -2.0 * b[c * _BC:(c + 1) * _BC, :]
        ab = jax.lax.dot_general(
            a, bc, (((1,), (1,)), ((), ())),
            preferred_element_type=jnp.float32,
        )
        for k in range(_BC // _L):
            j0 = c * _BC + k * _L
            t = b2r[:, j0:j0 + _L] + ab[:, k * _L:(k + 1) * _L]
            rowm = t if rowm is None else jnp.minimum(rowm, t)
        colc = jnp.min(a2 + ab, axis=0, keepdims=True)
        tot += jnp.sum(colc + b2r[:, c * _BC:(c + 1) * _BC])[None, None]

    rowfin = jnp.min(rowm, axis=1, keepdims=True)
    tot += jnp.sum(rowfin + a2)[None, None]

    @pl.when(n == 0)
    def _init():
        acc_ref[...] = jnp.zeros((1, 1), jnp.float32)

    acc_ref[...] += tot

    @pl.when(n == _N - 1)
    def _fin():
        out_ref[...] = acc_ref[...] * (1.0 / (_N * _P))


@jax.jit
def kernel(p1, p2):
    out = pl.pallas_call(
        _chamfer_kernel,
        grid=(_N,),
        in_specs=[
            pl.BlockSpec((1, _P, 3), lambda n: (n, 0, 0)),
            pl.BlockSpec((1, _P, 3), lambda n: (n, 0, 0)),
        ],
        out_specs=pl.BlockSpec((1, 1), lambda n: (0, 0)),
        out_shape=jax.ShapeDtypeStruct((1, 1), jnp.float32),
        scratch_shapes=[
            pltpu.VMEM((1, 1), jnp.float32),
        ],
        compiler_params=pltpu.CompilerParams(
            vmem_limit_bytes=100 * 1024 * 1024,
        ),
    )(p1, p2)
    return out[0, 0]
